# baseline (speedup 1.0000x reference)
import functools

import numpy as np
import jax
import jax.numpy as jnp
from jax import lax
from jax.experimental import pallas as pl
from jax.experimental.pallas import tpu as pltpu
from jax.experimental.pallas import tpu_sc as plsc

_B = 16384
_F = 26
_D = 16
_TOTAL = 2600000

_NCORES = 2
_NSUB = 16
_NW = _NCORES * _NSUB
_BPW = _B // _NW
_NC = 128
_CHUNKS = _BPW // _NC
_FP = 32
_CF = _NC * _FP
_NVR = _CF // 16
_PATV = 2
_GRP = _CF // 128

_CONST_NP = np.concatenate(
    [np.arange(_F, dtype=np.int32) * 100000, np.zeros(6, np.int32)])

_ROW_NP, _COL_NP = np.triu_indices(_F, k=1)


def _sc_body(x_hbm, offs_hbm, emb_hbm, lin_hbm, a_out, l_out,
             offs_v, xv, idxv, emb_q, lin_v, sem_e, sem_l):
    wid = lax.axis_index("s") * _NCORES + lax.axis_index("c")
    pltpu.sync_copy(offs_hbm, offs_v)

    def chunk(ci, carry):
        s0 = wid * _BPW + ci * _NC
        f0 = s0 * _FP
        pltpu.sync_copy(x_hbm.at[pl.ds(f0, _CF)], xv)
        for j in range(_NVR):
            idxv[pl.ds(j * 16, 16)] = (
                xv[pl.ds(j * 16, 16)] + offs_v[pl.ds((j % _PATV) * 16, 16)]
            )
        def gat(s, carry2):
            for k in range(4):
                pltpu.async_copy(
                    emb_hbm.at[idxv.at[pl.ds(s * _FP + 8 * k, 8)]],
                    emb_q.at[pl.ds(k * 1024 + s * 8, 8)], sem_e)
            return carry2
        lax.fori_loop(0, _NC, gat, 0)
        cps = []
        for g in range(_GRP):
            cps.append(pltpu.async_copy(
                lin_hbm.at[idxv.at[pl.ds(g * 128, 128)]],
                lin_v.at[pl.ds(g * 128, 128)], sem_l))
        def drain(s, carry2):
            pltpu.make_async_copy(
                emb_hbm.at[idxv.at[pl.ds(0, 8)]],
                emb_q.at[pl.ds(0, 8)], sem_e).wait()
            return carry2
        lax.fori_loop(0, 4 * _NC, drain, 0)
        for cp in cps:
            cp.wait()
        for k in range(4):
            pltpu.sync_copy(
                emb_q.at[pl.ds(k * 1024, 1024)],
                a_out.at[pl.ds((k * _B + s0) * 8, 1024)])
        pltpu.sync_copy(lin_v, l_out.at[pl.ds(f0, _CF)])
        return carry

    lax.fori_loop(0, _CHUNKS, chunk, 0)


def _sc_gather(x_flat, offs_pat, embed_table, lin_flat):
    mesh = plsc.VectorSubcoreMesh(
        core_axis_name="c", subcore_axis_name="s",
        num_cores=_NCORES, num_subcores=_NSUB)
    f = functools.partial(
        pl.kernel,
        out_type=[
            jax.ShapeDtypeStruct((4 * _B * 8, _D), jnp.float32),
            jax.ShapeDtypeStruct((_B * _FP,), jnp.float32),
        ],
        mesh=mesh,
        scratch_types=[
            pltpu.VMEM((_CONST_NP.size,), jnp.int32),
            pltpu.VMEM((_CF,), jnp.int32),
            pltpu.VMEM((_CF,), jnp.int32),
            pltpu.VMEM((4 * 1024, _D), jnp.float32),
            pltpu.VMEM((_CF,), jnp.float32),
            pltpu.SemaphoreType.DMA,
            pltpu.SemaphoreType.DMA,
        ],
        compiler_params=pltpu.CompilerParams(use_tc_tiling_on_sc=False),
    )(_sc_body)
    return f(x_flat, offs_pat, embed_table, lin_flat)


_BLK = 512
_NBLK = _B // _BLK


def _tc_body(a0_ref, a1_ref, a2_ref, a3_ref, l_ref, w_ref, b_ref, o_ref):
    a = jnp.concatenate(
        [a0_ref[...], a1_ref[...], a2_ref[...], a3_ref[...]], axis=1)
    y = jnp.dot(a, w_ref[...], preferred_element_type=jnp.float32)
    s = jnp.sum(a * y, axis=1, keepdims=True)
    s = s + jnp.sum(l_ref[...], axis=1, keepdims=True)
    o_ref[...] = s + b_ref[0, 0]


def _tc_interact(aq, lg2d, wkp, bias2):
    aspec = lambda k: pl.BlockSpec((_BLK, 128), lambda i, k=k: (k * _NBLK + i, 0))
    return pl.pallas_call(
        _tc_body,
        grid=(_NBLK,),
        in_specs=[
            aspec(0), aspec(1), aspec(2), aspec(3),
            pl.BlockSpec((_BLK, _FP), lambda i: (i, 0)),
            pl.BlockSpec((512, 512), lambda i: (0, 0)),
            pl.BlockSpec(memory_space=pltpu.SMEM),
        ],
        out_specs=pl.BlockSpec((_BLK, 1), lambda i: (i, 0)),
        out_shape=jax.ShapeDtypeStruct((_B, 1), jnp.float32),
    )(aq, aq, aq, aq, lg2d, wkp, bias2)


def kernel(x, embed_table, linear_table, linear_bias, fwfm_W, fwfm_b):
    xpad = jnp.pad(x.astype(jnp.int32), ((0, 0), (0, _FP - _F)))
    x_flat = xpad.reshape(-1)
    offs_pat = jnp.asarray(_CONST_NP)
    lin_flat = lax.squeeze(linear_table, (1,))
    a_flat, l_flat = _sc_gather(x_flat, offs_pat, embed_table, lin_flat)
    aq = a_flat.reshape(4 * _B, 8 * _D)
    lg2d = l_flat.reshape(_B, _FP)
    w = fwfm_W[:, 0]
    wm = jnp.zeros((32, 32), jnp.float32).at[_ROW_NP, _COL_NP].set(w)
    wsym = wm + wm.T
    wkp = 0.5 * jnp.kron(wsym, jnp.eye(_D, dtype=jnp.float32))
    bias2 = (linear_bias[0] + fwfm_b[0]
             - (_FP - _F) * lin_flat[0]).reshape(1, 1)
    return _tc_interact(aq, lg2d, wkp, bias2)


# speedup vs baseline: 1.5161x; 1.5161x over previous
import functools

import numpy as np
import jax
import jax.numpy as jnp
from jax import lax
from jax.experimental import pallas as pl
from jax.experimental.pallas import tpu as pltpu
from jax.experimental.pallas import tpu_sc as plsc

_B = 16384
_F = 26
_D = 16
_TOTAL = 2600000

_NCORES = 2
_NSUB = 16
_NW = _NCORES * _NSUB
_BPW = _B // _NW
_NC = 128
_CHUNKS = _BPW // _NC
_CF = _NC * _F
_NVR = _CF // 16
_PATV = 13
_GRP = _CF // 128

_CONST_NP = np.tile(np.arange(_F, dtype=np.int32) * 100000, 8)

_ROW_NP, _COL_NP = np.triu_indices(_F, k=1)


def _sc_body(x_hbm, offs_hbm, emb_hbm, lin_hbm, a_out, l_out,
             offs_v, xv, idxv, emb_v, lin_v, sem_e, sem_l):
    wid = lax.axis_index("s") * _NCORES + lax.axis_index("c")
    pltpu.sync_copy(offs_hbm, offs_v)

    def chunk(ci, carry):
        f0 = (wid * _BPW + ci * _NC) * _F
        pltpu.sync_copy(x_hbm.at[pl.ds(f0, _CF)], xv)
        for j in range(_NVR):
            idxv[pl.ds(j * 16, 16)] = (
                xv[pl.ds(j * 16, 16)] + offs_v[pl.ds((j % _PATV) * 16, 16)]
            )
        cps = []
        for g in range(_GRP):
            cps.append(pltpu.async_copy(
                emb_hbm.at[idxv.at[pl.ds(g * 128, 128)]],
                emb_v.at[pl.ds(g * 128, 128)], sem_e))
            cps.append(pltpu.async_copy(
                lin_hbm.at[idxv.at[pl.ds(g * 128, 128)]],
                lin_v.at[pl.ds(g * 128, 128)], sem_l))
        for cp in cps:
            cp.wait()
        pltpu.sync_copy(emb_v, a_out.at[pl.ds(f0, _CF)])
        pltpu.sync_copy(lin_v, l_out.at[pl.ds(f0, _CF)])
        return carry

    lax.fori_loop(0, _CHUNKS, chunk, 0)


def _sc_gather(x_flat, offs_pat, embed_table, lin_flat):
    mesh = plsc.VectorSubcoreMesh(
        core_axis_name="c", subcore_axis_name="s",
        num_cores=_NCORES, num_subcores=_NSUB)
    f = functools.partial(
        pl.kernel,
        out_type=[
            jax.ShapeDtypeStruct((_B * _F, _D), jnp.float32),
            jax.ShapeDtypeStruct((_B * _F,), jnp.float32),
        ],
        mesh=mesh,
        scratch_types=[
            pltpu.VMEM((_CONST_NP.size,), jnp.int32),
            pltpu.VMEM((_CF,), jnp.int32),
            pltpu.VMEM((_CF,), jnp.int32),
            pltpu.VMEM((_CF, _D), jnp.float32),
            pltpu.VMEM((_CF,), jnp.float32),
            pltpu.SemaphoreType.DMA,
            pltpu.SemaphoreType.DMA,
        ],
        compiler_params=pltpu.CompilerParams(use_tc_tiling_on_sc=False),
    )(_sc_body)
    return f(x_flat, offs_pat, embed_table, lin_flat)


_BLK = 512
_NBLK = _B // _BLK


def _tc_body(a_ref, l_ref, w_ref, b_ref, o_ref):
    a = a_ref[...]
    y = jnp.dot(a, w_ref[...], preferred_element_type=jnp.float32)
    s = jnp.sum(a * y, axis=1, keepdims=True)
    s = s + jnp.sum(l_ref[...], axis=1, keepdims=True)
    o_ref[...] = s + b_ref[0, 0]


def _tc_interact(a2d, lg2d, wk, bias2):
    return pl.pallas_call(
        _tc_body,
        grid=(_NBLK,),
        in_specs=[
            pl.BlockSpec((_BLK, _F * _D), lambda i: (i, 0)),
            pl.BlockSpec((_BLK, _F), lambda i: (i, 0)),
            pl.BlockSpec((_F * _D, _F * _D), lambda i: (0, 0)),
            pl.BlockSpec(memory_space=pltpu.SMEM),
        ],
        out_specs=pl.BlockSpec((_BLK, 1), lambda i: (i, 0)),
        out_shape=jax.ShapeDtypeStruct((_B, 1), jnp.float32),
    )(a2d, lg2d, wk, bias2)


def kernel(x, embed_table, linear_table, linear_bias, fwfm_W, fwfm_b):
    x_flat = x.reshape(-1).astype(jnp.int32)
    offs_pat = jnp.asarray(_CONST_NP)
    lin_flat = lax.squeeze(linear_table, (1,))
    a_flat, l_flat = _sc_gather(x_flat, offs_pat, embed_table, lin_flat)
    a2d = a_flat.reshape(_B, _F * _D)
    lg2d = l_flat.reshape(_B, _F)
    w = fwfm_W[:, 0]
    wm = jnp.zeros((_F, _F), jnp.float32).at[_ROW_NP, _COL_NP].set(w)
    wsym = wm + wm.T
    wk = 0.5 * jnp.kron(wsym, jnp.eye(_D, dtype=jnp.float32))
    bias2 = (linear_bias[0] + fwfm_b[0]).reshape(1, 1)
    return _tc_interact(a2d, lg2d, wk, bias2)
